# baseline (speedup 1.0000x reference)
import functools

import jax
import jax.numpy as jnp
from jax import lax
from jax.experimental import pallas as pl
from jax.experimental.pallas import tpu as pltpu
from jax.experimental.pallas import tpu_sc as plsc

_B, _M, _S, _E, _V = 1024, 50, 20, 32, 100000
_NW = 32
_CR = 500
_NC = (_B * _M * _S) // (_NW * _CR)
_TOTC = _B * _M * _S // _CR
_JG = 4
_GSZ = _CR // _JG
_NBUF = 4
_CL = _CR * _E // 128
_BB = 16
_R128 = _M * _S * _E // 128


def _position_encoding(sent_size, emb_size):
    j = jnp.arange(1, sent_size + 1, dtype=jnp.float32)[:, None]
    k = jnp.arange(1, emb_size + 1, dtype=jnp.float32)[None, :]
    return (1.0 - j / sent_size) - (k / emb_size) * (1.0 - 2.0 * j / sent_size)


def _sc_body(x_hbm, w_hbm, g_hbm, idx_v, gbuf_v, sbuf_v, isems, gsems, ssems):
    wid = lax.axis_index("s") * 2 + lax.axis_index("c")
    c_base = wid * _NC

    def issue_idx(c, b):
        pltpu.async_copy(x_hbm.at[c_base + c], idx_v.at[b], isems[b])

    def wait_idx(b):
        pltpu.make_async_copy(x_hbm.at[0], idx_v.at[b], isems[b]).wait()

    def issue_gathers(c, b):
        for j in range(_JG):
            pltpu.async_copy(w_hbm.at[idx_v.at[b, j]],
                             gbuf_v.at[b, pl.ds(j * _GSZ, _GSZ)], gsems[b])

    def wait_gathers(b):
        pltpu.make_async_copy(w_hbm.at[pl.ds(0, _CR)], gbuf_v.at[b],
                              gsems[b]).wait()

    def issue_scatter(c, b):
        pltpu.async_copy(sbuf_v.at[b],
                         g_hbm.at[pl.ds((c_base + c) * _CL, _CL)], ssems[b])

    def wait_scatter(c, b):
        pltpu.make_async_copy(sbuf_v.at[b],
                              g_hbm.at[pl.ds((c_base + c) * _CL, _CL)],
                              ssems[b]).wait()

    def repack(b):
        def line_body(j, _):
            for k in range(4):
                sbuf_v[b, j, pl.ds(32 * k, 32)] = gbuf_v[b, 4 * j + k, :]
            return 0

        lax.fori_loop(0, _CL, line_body, 0)

    for c0 in range(_NBUF):
        issue_idx(c0, c0)
    wait_idx(0)
    issue_gathers(0, 0)

    def phase(c, b):
        b1 = (b + 1) % _NBUF

        wait_gathers(b)

        @pl.when(c + _NBUF < _NC)
        def _():
            issue_idx(c + _NBUF, b)

        @pl.when(c + 1 < _NC)
        def _():
            wait_idx(b1)
            issue_gathers(c + 1, b1)

        @pl.when(c >= _NBUF)
        def _():
            wait_scatter(c - _NBUF, b)

        repack(b)
        issue_scatter(c, b)

    def chunk_body(t, _):
        for jb in range(_NBUF):
            phase(t * _NBUF + jb, jb)
        return 0

    lax.fori_loop(0, _NC // _NBUF, chunk_body, 0)

    for c in range(_NC - _NBUF, _NC):
        wait_scatter(c, c % _NBUF)


def _tc_body(g_ref, pef_ref, tff_ref, out_ref):
    v = g_ref[...].reshape(_BB, _R128, 128).astype(jnp.float32)
    out_ref[...] = v * pef_ref[None] + tff_ref[None]


@jax.jit
def kernel(x, W_emb, W_temp):
    pe = _position_encoding(_S, _E)
    x3 = x.reshape(_TOTC, _JG, _GSZ).astype(jnp.int32)
    w_bf = W_emb.astype(jnp.bfloat16)

    mesh = plsc.VectorSubcoreMesh(core_axis_name="c", subcore_axis_name="s")
    gather_rows = pl.kernel(
        _sc_body,
        out_type=jax.ShapeDtypeStruct((_TOTC * _CL, 128), jnp.bfloat16),
        mesh=mesh,
        scratch_types=[
            pltpu.VMEM((_NBUF, _JG, _GSZ), jnp.int32),
            pltpu.VMEM((_NBUF, _CR, _E), jnp.bfloat16),
            pltpu.VMEM((_NBUF, _CL, 128), jnp.bfloat16),
            [pltpu.SemaphoreType.DMA] * _NBUF,
            [pltpu.SemaphoreType.DMA] * _NBUF,
            [pltpu.SemaphoreType.DMA] * _NBUF,
        ],
        compiler_params=pltpu.CompilerParams(use_tc_tiling_on_sc=False),
    )
    g = gather_rows(x3, w_bf)

    pef = jnp.broadcast_to(pe[None, :, :], (_M, _S, _E)).reshape(-1, 128)
    tff = jnp.broadcast_to(W_temp[:, None, :], (_M, _S, _E)).reshape(-1, 128)

    out = pl.pallas_call(
        _tc_body,
        out_shape=jax.ShapeDtypeStruct((_B, _R128, 128), jnp.float32),
        grid=(_B // _BB,),
        in_specs=[
            pl.BlockSpec((_BB * _R128, 128), lambda i: (i, 0)),
            pl.BlockSpec((_R128, 128), lambda i: (0, 0)),
            pl.BlockSpec((_R128, 128), lambda i: (0, 0)),
        ],
        out_specs=pl.BlockSpec((_BB, _R128, 128), lambda i: (i, 0, 0)),
    )(g, pef, tff)
    return out.reshape(_B, _M, _S, _E)


# speedup vs baseline: 1.0489x; 1.0489x over previous
import functools

import jax
import jax.numpy as jnp
from jax import lax
from jax.experimental import pallas as pl
from jax.experimental.pallas import tpu as pltpu
from jax.experimental.pallas import tpu_sc as plsc

_B, _M, _S, _E, _V = 1024, 50, 20, 32, 100000
_NW = 32
_CR = 640
_NC = (_B * _M * _S) // (_NW * _CR)
_TOTC = _B * _M * _S // _CR
_JG = 8
_GSZ = _CR // _JG
_NBUF = 4
_CL = _CR * _E // 256
_BB = 16
_R128 = _M * _S * _E // 128
_RI32 = _M * _S * _E // 256


def _position_encoding(sent_size, emb_size):
    j = jnp.arange(1, sent_size + 1, dtype=jnp.float32)[:, None]
    k = jnp.arange(1, emb_size + 1, dtype=jnp.float32)[None, :]
    return (1.0 - j / sent_size) - (k / emb_size) * (1.0 - 2.0 * j / sent_size)


def _sc_body(x_hbm, w_hbm, g_hbm, idx_v, gbuf_v, sbuf_v, isems, gsems, ssems):
    wid = lax.axis_index("s") * 2 + lax.axis_index("c")
    c_base = wid * _NC

    def issue_idx(c, b):
        pltpu.async_copy(x_hbm.at[c_base + c], idx_v.at[b], isems[b])

    def wait_idx(b):
        pltpu.make_async_copy(x_hbm.at[0], idx_v.at[b], isems[b]).wait()

    def issue_gathers(c, b):
        for j in range(_JG):
            pltpu.async_copy(w_hbm.at[idx_v.at[b, j]],
                             gbuf_v.at[b, pl.ds(j * _GSZ, _GSZ)], gsems[b])

    def wait_gathers(b):
        pltpu.make_async_copy(w_hbm.at[pl.ds(0, _CR)], gbuf_v.at[b],
                              gsems[b]).wait()

    def issue_scatter(c, b):
        pltpu.async_copy(sbuf_v.at[b],
                         g_hbm.at[pl.ds((c_base + c) * _CL, _CL)], ssems[b])

    def wait_scatter(c, b):
        pltpu.make_async_copy(sbuf_v.at[b],
                              g_hbm.at[pl.ds((c_base + c) * _CL, _CL)],
                              ssems[b]).wait()

    def repack(b):
        def line_body(j, _):
            for k in range(8):
                sbuf_v[b, j, pl.ds(16 * k, 16)] = plsc.bitcast(
                    gbuf_v[b, 8 * j + k, :], jnp.int32)
            return 0

        lax.fori_loop(0, _CL, line_body, 0)

    for c0 in range(_NBUF):
        issue_idx(c0, c0)
    wait_idx(0)
    issue_gathers(0, 0)

    def phase(c, b):
        b1 = (b + 1) % _NBUF

        wait_gathers(b)

        @pl.when(c + _NBUF < _NC)
        def _():
            issue_idx(c + _NBUF, b)

        @pl.when(c + 1 < _NC)
        def _():
            wait_idx(b1)
            issue_gathers(c + 1, b1)

        @pl.when(c >= _NBUF)
        def _():
            wait_scatter(c - _NBUF, b)

        repack(b)
        issue_scatter(c, b)

    def chunk_body(t, _):
        for jb in range(_NBUF):
            phase(t * _NBUF + jb, jb)
        return 0

    lax.fori_loop(0, _NC // _NBUF, chunk_body, 0)
    for c in range((_NC // _NBUF) * _NBUF, _NC):
        phase(c, c % _NBUF)

    for c in range(_NC - _NBUF, _NC):
        wait_scatter(c, c % _NBUF)


def _tc_body(g_ref, pef_ref, tff_ref, out_ref):
    w = g_ref[...].reshape(_BB, _RI32, 128)
    lo = lax.bitcast_convert_type(w << 16, jnp.float32)
    hi = lax.bitcast_convert_type(w & jnp.int32(-65536), jnp.float32)
    parts = []
    for gidx in range(8):
        parts.append(lo[:, :, 16 * gidx:16 * (gidx + 1)])
        parts.append(hi[:, :, 16 * gidx:16 * (gidx + 1)])
    v = jnp.concatenate(parts, axis=-1)
    out_ref[...] = v * pef_ref[None] + tff_ref[None]


@jax.jit
def kernel(x, W_emb, W_temp):
    pe = _position_encoding(_S, _E)
    x3 = x.reshape(_TOTC, _JG, _GSZ).astype(jnp.int32)
    perm = jnp.stack([jnp.arange(16), jnp.arange(16) + 16], axis=1).reshape(-1)
    w_bf = W_emb[:, perm].astype(jnp.bfloat16)

    mesh = plsc.VectorSubcoreMesh(core_axis_name="c", subcore_axis_name="s")
    gather_rows = pl.kernel(
        _sc_body,
        out_type=jax.ShapeDtypeStruct((_TOTC * _CL, 128), jnp.int32),
        mesh=mesh,
        scratch_types=[
            pltpu.VMEM((_NBUF, _JG, _GSZ), jnp.int32),
            pltpu.VMEM((_NBUF, _CR, _E), jnp.bfloat16),
            pltpu.VMEM((_NBUF, _CL, 128), jnp.int32),
            [pltpu.SemaphoreType.DMA] * _NBUF,
            [pltpu.SemaphoreType.DMA] * _NBUF,
            [pltpu.SemaphoreType.DMA] * _NBUF,
        ],
        compiler_params=pltpu.CompilerParams(use_tc_tiling_on_sc=False,
                                             needs_layout_passes=False),
    )
    g = gather_rows(x3, w_bf)

    pef = jnp.broadcast_to(pe[None, :, :], (_M, _S, _E)).reshape(_RI32, 256)
    tff = jnp.broadcast_to(W_temp[:, None, :], (_M, _S, _E)).reshape(_RI32, 256)

    out = pl.pallas_call(
        _tc_body,
        out_shape=jax.ShapeDtypeStruct((_B, _RI32, 256), jnp.float32),
        grid=(_B // _BB,),
        in_specs=[
            pl.BlockSpec((_BB * _RI32, 128), lambda i: (i, 0)),
            pl.BlockSpec((_RI32, 256), lambda i: (0, 0)),
            pl.BlockSpec((_RI32, 256), lambda i: (0, 0)),
        ],
        out_specs=pl.BlockSpec((_BB, _RI32, 256), lambda i: (i, 0, 0)),
    )(g, pef, tff)
    return out.reshape(_B, _M, _S, _E)


# speedup vs baseline: 1.0843x; 1.0338x over previous
import functools

import jax
import jax.numpy as jnp
from jax import lax
from jax.experimental import pallas as pl
from jax.experimental.pallas import tpu as pltpu
from jax.experimental.pallas import tpu_sc as plsc

_B, _M, _S, _E, _V = 1024, 50, 20, 32, 100000
_NW = 32
_CR = 500
_NC = (_B * _M * _S) // (_NW * _CR)
_TOTC = _B * _M * _S // _CR
_JG = 4
_GSZ = _CR // _JG
_NBUF = 4
_BB = 16
_R128 = _M * _S * _E // 128


def _position_encoding(sent_size, emb_size):
    j = jnp.arange(1, sent_size + 1, dtype=jnp.float32)[:, None]
    k = jnp.arange(1, emb_size + 1, dtype=jnp.float32)[None, :]
    return (1.0 - j / sent_size) - (k / emb_size) * (1.0 - 2.0 * j / sent_size)


def _sc_body(x_hbm, w_hbm, g_hbm, idx_v, bbuf_v, isems, gsems, ssems):
    wid = lax.axis_index("s") * 2 + lax.axis_index("c")
    c_base = wid * _NC

    def issue_idx(c, b):
        pltpu.async_copy(x_hbm.at[c_base + c], idx_v.at[b], isems[b])

    def wait_idx(b):
        pltpu.make_async_copy(x_hbm.at[0], idx_v.at[b], isems[b]).wait()

    def issue_gathers(c, b):
        for j in range(_JG):
            pltpu.async_copy(w_hbm.at[idx_v.at[b, j]],
                             bbuf_v.at[b, pl.ds(j * _GSZ, _GSZ)], gsems[b])

    def wait_gathers(b):
        pltpu.make_async_copy(w_hbm.at[pl.ds(0, _CR)], bbuf_v.at[b],
                              gsems[b]).wait()

    def wait_scatter(c, b):
        pltpu.make_async_copy(bbuf_v.at[b], g_hbm.at[c_base + c],
                              ssems[b]).wait()

    for c0 in range(_NBUF):
        issue_idx(c0, c0)
    wait_idx(0)
    issue_gathers(0, 0)

    def phase(c, b):
        b1 = (b + 1) % _NBUF

        wait_gathers(b)

        @pl.when(c + _NBUF < _NC)
        def _():
            issue_idx(c + _NBUF, b)

        pltpu.async_copy(bbuf_v.at[b], g_hbm.at[c_base + c], ssems[b])

        @pl.when(c + 1 < _NC)
        def _():
            wait_idx(b1)

            @pl.when(c + 1 >= _NBUF)
            def _():
                wait_scatter(c + 1 - _NBUF, b1)

            issue_gathers(c + 1, b1)

    def chunk_body(t, _):
        for jb in range(_NBUF):
            phase(t * _NBUF + jb, jb)
        return 0

    lax.fori_loop(0, _NC // _NBUF, chunk_body, 0)

    for c in range(_NC - _NBUF, _NC):
        wait_scatter(c, c % _NBUF)


def _tc_body(g_ref, pef_ref, tff_ref, out_ref):
    out_ref[...] = (g_ref[...].astype(jnp.float32) * pef_ref[None]
                    + tff_ref[None])


@jax.jit
def kernel(x, W_emb, W_temp):
    pe = _position_encoding(_S, _E)
    x3 = x.reshape(_TOTC, _JG, _GSZ).astype(jnp.int32)
    w_bf = W_emb.astype(jnp.bfloat16)

    mesh = plsc.VectorSubcoreMesh(core_axis_name="c", subcore_axis_name="s")
    gather_rows = pl.kernel(
        _sc_body,
        out_type=jax.ShapeDtypeStruct((_TOTC, _CR, _E), jnp.bfloat16),
        mesh=mesh,
        scratch_types=[
            pltpu.VMEM((_NBUF, _JG, _GSZ), jnp.int32),
            pltpu.VMEM((_NBUF, _CR, _E), jnp.bfloat16),
            [pltpu.SemaphoreType.DMA] * _NBUF,
            [pltpu.SemaphoreType.DMA] * _NBUF,
            [pltpu.SemaphoreType.DMA] * _NBUF,
        ],
        compiler_params=pltpu.CompilerParams(use_tc_tiling_on_sc=False),
    )
    g = gather_rows(x3, w_bf).reshape(_B, _R128, 128)

    pef = jnp.broadcast_to(pe[None, :, :], (_M, _S, _E)).reshape(-1, 128)
    tff = jnp.broadcast_to(W_temp[:, None, :], (_M, _S, _E)).reshape(-1, 128)

    out = pl.pallas_call(
        _tc_body,
        out_shape=jax.ShapeDtypeStruct((_B, _R128, 128), jnp.float32),
        grid=(_B // _BB,),
        in_specs=[
            pl.BlockSpec((_BB, _R128, 128), lambda i: (i, 0, 0)),
            pl.BlockSpec((_R128, 128), lambda i: (0, 0)),
            pl.BlockSpec((_R128, 128), lambda i: (0, 0)),
        ],
        out_specs=pl.BlockSpec((_BB, _R128, 128), lambda i: (i, 0, 0)),
    )(g, pef, tff)
    return out.reshape(_B, _M, _S, _E)
